# Initial kernel scaffold; baseline (speedup 1.0000x reference)
import jax
import jax.numpy as jnp
from jax.experimental import pallas as pl


def kernel(z, codebook):
    raise NotImplementedError("write your pallas kernel here")


# speedup vs baseline: 3.9960x; 3.9960x over previous
import functools

import jax
import jax.numpy as jnp
from jax.experimental import pallas as pl
from jax.experimental.pallas import tpu as pltpu

NUM_CODEBOOKS = 8
CODEBOOK_SIZE = 512
CODE_DIM = 64
BATCH = 1024


def _vq_kernel(z_ref, cb_ref, zq_ref, idx_ref, counts_ref, scal_ref,
               commit_smem):
    g = pl.program_id(0)
    z = z_ref[0]
    c = cb_ref[0]

    c_sq = jnp.sum(c * c, axis=1, keepdims=True)
    cross = jax.lax.dot_general(
        z, c, (((1,), (1,)), ((), ())),
        preferred_element_type=jnp.float32)
    d = jnp.transpose(c_sq) - 2.0 * cross

    dmin = jnp.min(d, axis=1, keepdims=True)
    k_iota = jax.lax.broadcasted_iota(jnp.int32, (BATCH, CODEBOOK_SIZE), 1)
    idx_col = jnp.min(jnp.where(d <= dmin, k_iota, CODEBOOK_SIZE),
                      axis=1, keepdims=True)
    one_hot = (k_iota == idx_col).astype(jnp.float32)

    zq = jax.lax.dot_general(
        one_hot, c, (((1,), (0,)), ((), ())),
        preferred_element_type=jnp.float32)
    zq_ref[0] = zq
    idx_ref[0] = jnp.reshape(idx_col, (1, BATCH))

    diff = z - zq
    csum = jnp.sum(diff * diff)
    cnt = jnp.sum(one_hot, axis=0, keepdims=True)

    @pl.when(g == 0)
    def _init():
        counts_ref[...] = jnp.zeros_like(counts_ref)
        commit_smem[0] = 0.0

    counts_ref[...] += cnt
    commit_smem[0] += csum

    @pl.when(g == NUM_CODEBOOKS - 1)
    def _finish():
        usage = counts_ref[...] / float(BATCH * NUM_CODEBOOKS)
        ent = -jnp.sum(usage * jnp.log(usage + 1e-10))
        commit = commit_smem[0] / float(BATCH * NUM_CODEBOOKS * CODE_DIM)
        lane = jax.lax.broadcasted_iota(jnp.int32, (1, 128), 1)
        out = jnp.where(lane == 0, commit,
                        jnp.where(lane == 1, ent, jnp.exp(ent)))
        scal_ref[...] = out


@functools.partial(jax.jit, static_argnames=())
def kernel(z, codebook):
    zt = jnp.transpose(
        z.reshape(BATCH, NUM_CODEBOOKS, CODE_DIM), (1, 0, 2))

    grid = (NUM_CODEBOOKS,)
    out_shapes = (
        jax.ShapeDtypeStruct((NUM_CODEBOOKS, BATCH, CODE_DIM), jnp.float32),
        jax.ShapeDtypeStruct((NUM_CODEBOOKS, 1, BATCH), jnp.int32),
        jax.ShapeDtypeStruct((1, CODEBOOK_SIZE), jnp.float32),
        jax.ShapeDtypeStruct((1, 128), jnp.float32),
    )
    zq_t, idx_t, _counts, scal = pl.pallas_call(
        _vq_kernel,
        grid=grid,
        in_specs=[
            pl.BlockSpec((1, BATCH, CODE_DIM), lambda g: (g, 0, 0)),
            pl.BlockSpec((1, CODEBOOK_SIZE, CODE_DIM), lambda g: (g, 0, 0)),
        ],
        out_specs=(
            pl.BlockSpec((1, BATCH, CODE_DIM), lambda g: (g, 0, 0)),
            pl.BlockSpec((1, 1, BATCH), lambda g: (g, 0, 0)),
            pl.BlockSpec((1, CODEBOOK_SIZE), lambda g: (0, 0)),
            pl.BlockSpec((1, 128), lambda g: (0, 0)),
        ),
        out_shape=out_shapes,
        scratch_shapes=[pltpu.SMEM((1,), jnp.float32)],
    )(zt, codebook)

    quantized = jnp.transpose(zq_t, (1, 0, 2)).reshape(
        BATCH, NUM_CODEBOOKS * CODE_DIM)
    indices = jnp.transpose(idx_t.reshape(NUM_CODEBOOKS, BATCH), (1, 0))
    commitment_loss = scal[0, 0]
    codebook_loss = jnp.zeros((), dtype=jnp.float32)
    entropy = scal[0, 1]
    perplexity = scal[0, 2]
    return (quantized, indices, commitment_loss, codebook_loss,
            entropy, perplexity)


# speedup vs baseline: 5.7201x; 1.4315x over previous
import jax
import jax.numpy as jnp
from jax.experimental import pallas as pl
from jax.experimental.pallas import tpu as pltpu

NUM_CODEBOOKS = 8
CODEBOOK_SIZE = 512
CODE_DIM = 64
BATCH = 1024
_PAIRS = NUM_CODEBOOKS // 2


def _vq_kernel(z_ref, cb_ref, zq_ref, idx_ref, counts_ref, scal_ref,
               commit_smem):
    j = pl.program_id(0)
    z2 = z_ref[...]

    @pl.when(j == 0)
    def _init():
        counts_ref[...] = jnp.zeros_like(counts_ref)
        commit_smem[0] = 0.0

    kf = jax.lax.broadcasted_iota(
        jnp.int32, (BATCH, CODEBOOK_SIZE), 1).astype(jnp.float32)
    lane_g = jax.lax.broadcasted_iota(jnp.int32, (BATCH, NUM_CODEBOOKS), 1)

    zq_halves = []
    cnt_total = None
    csum = jnp.sum(z2 * z2)
    for h in range(2):
        zg = z2[:, h * CODE_DIM:(h + 1) * CODE_DIM]
        c = cb_ref[h]
        c_sq = jnp.sum(c * c, axis=1, keepdims=True)
        cross = jax.lax.dot_general(
            zg, c, (((1,), (1,)), ((), ())),
            preferred_element_type=jnp.float32)
        d = jnp.transpose(c_sq) - 2.0 * cross
        dmin = jnp.min(d, axis=1, keepdims=True)
        idx_f = jnp.min(jnp.where(d <= dmin, kf, float(CODEBOOK_SIZE)),
                        axis=1, keepdims=True)
        one_hot = (kf == idx_f).astype(jnp.float32)
        zq_halves.append(jax.lax.dot_general(
            one_hot, c, (((1,), (0,)), ((), ())),
            preferred_element_type=jnp.float32))

        cnt = jnp.sum(one_hot, axis=0, keepdims=True)
        cnt_total = cnt if cnt_total is None else cnt_total + cnt
        csum = csum + jnp.sum(dmin)
        idx_col = idx_f.astype(jnp.int32)
        idx_ref[...] = jnp.where(lane_g == 2 * j + h, idx_col, idx_ref[...])

    zq_ref[...] = jnp.concatenate(zq_halves, axis=1)
    counts_ref[...] += cnt_total
    commit_smem[0] += csum

    @pl.when(j == _PAIRS - 1)
    def _finish():
        usage = counts_ref[...] / float(BATCH * NUM_CODEBOOKS)
        ent = -jnp.sum(usage * jnp.log(usage + 1e-10))
        commit = commit_smem[0] / float(BATCH * NUM_CODEBOOKS * CODE_DIM)
        lane = jax.lax.broadcasted_iota(jnp.int32, (1, 128), 1)
        out = jnp.where(lane == 0, commit,
                        jnp.where(lane == 1, ent, jnp.exp(ent)))
        scal_ref[...] = out


@jax.jit
def kernel(z, codebook):
    grid = (_PAIRS,)
    out_shapes = (
        jax.ShapeDtypeStruct((BATCH, NUM_CODEBOOKS * CODE_DIM), jnp.float32),
        jax.ShapeDtypeStruct((BATCH, NUM_CODEBOOKS), jnp.int32),
        jax.ShapeDtypeStruct((1, CODEBOOK_SIZE), jnp.float32),
        jax.ShapeDtypeStruct((1, 128), jnp.float32),
    )
    quantized, indices, _counts, scal = pl.pallas_call(
        _vq_kernel,
        grid=grid,
        in_specs=[
            pl.BlockSpec((BATCH, 2 * CODE_DIM), lambda j: (0, j)),
            pl.BlockSpec((2, CODEBOOK_SIZE, CODE_DIM), lambda j: (j, 0, 0)),
        ],
        out_specs=(
            pl.BlockSpec((BATCH, 2 * CODE_DIM), lambda j: (0, j)),
            pl.BlockSpec((BATCH, NUM_CODEBOOKS), lambda j: (0, 0)),
            pl.BlockSpec((1, CODEBOOK_SIZE), lambda j: (0, 0)),
            pl.BlockSpec((1, 128), lambda j: (0, 0)),
        ),
        out_shape=out_shapes,
        scratch_shapes=[pltpu.SMEM((1,), jnp.float32)],
    )(z, codebook)

    commitment_loss = scal[0, 0]
    codebook_loss = jnp.zeros((), dtype=jnp.float32)
    entropy = scal[0, 1]
    perplexity = scal[0, 2]
    return (quantized, indices, commitment_loss, codebook_loss,
            entropy, perplexity)
